# Initial kernel scaffold; baseline (speedup 1.0000x reference)
import jax
import jax.numpy as jnp
from jax.experimental import pallas as pl


def kernel(x, edge_index, W1, b1, W2, b2, Wo, bo):
    raise NotImplementedError("write your pallas kernel here")


# speedup vs baseline: 6.8421x; 6.8421x over previous
import functools

import jax
import jax.numpy as jnp
from jax import lax
from jax.experimental import pallas as pl
from jax.experimental.pallas import tpu as pltpu
from jax.experimental.pallas import tpu_sc as plsc

N = 10000
D = 128
E = 320000
CHUNK = 128
CHUNKS = 2560
E_PAD = CHUNKS * CHUNK
N_PAD = 10112
NC, NS = 2, 16
NW = NC * NS
CPW = CHUNKS // NW
RPT = N_PAD // NS


def _sc_agg_body(h_hbm, src_hbm, dst_hbm, zeros_hbm, out_hbm,
                 sidx, didx, rows, acc, sem):
    c = lax.axis_index("c")
    s = lax.axis_index("s")
    wid = c * NS + s
    r0 = s * RPT
    pltpu.sync_copy(zeros_hbm.at[pl.ds(r0, RPT)], acc.at[pl.ds(r0, RPT)])
    plsc.subcore_barrier()

    @pl.loop(0, CPW)
    def _(i):
        e0 = (wid * CPW + i) * CHUNK
        pltpu.sync_copy(src_hbm.at[pl.ds(e0, CHUNK)], sidx)
        pltpu.sync_copy(dst_hbm.at[pl.ds(e0, CHUNK)], didx)
        pltpu.async_copy(h_hbm.at[sidx], rows, sem).wait()
        pltpu.sync_copy(rows, acc.at[didx], add=True)

    plsc.subcore_barrier()
    pltpu.sync_copy(acc.at[pl.ds(r0, RPT)], out_hbm.at[c].at[pl.ds(r0, RPT)])


def _make_sc_agg(width):
    mesh = plsc.VectorSubcoreMesh(core_axis_name="c", subcore_axis_name="s")
    return pl.kernel(
        _sc_agg_body,
        out_type=jax.ShapeDtypeStruct((NC, N_PAD, width), jnp.float32),
        mesh=mesh,
        scratch_types=[
            pltpu.VMEM((CHUNK,), jnp.int32),
            pltpu.VMEM((CHUNK,), jnp.int32),
            pltpu.VMEM((CHUNK, width), jnp.float32),
            pltpu.VMEM_SHARED((N_PAD, width), jnp.float32),
            pltpu.SemaphoreType.DMA,
        ],
    )


_sc_agg128 = _make_sc_agg(D)


def _sc_deg_body(dst_hbm, out_hbm, didx, deg_local, tmp_v, acc_v, shared):
    c = lax.axis_index("c")
    s = lax.axis_index("s")
    wid = c * NS + s

    @pl.loop(0, N_PAD, step=16)
    def _(j):
        deg_local[pl.ds(j, 16)] = jnp.zeros((16,), jnp.float32)

    @pl.loop(0, CPW)
    def _(i):
        e0 = (wid * CPW + i) * CHUNK
        pltpu.sync_copy(dst_hbm.at[pl.ds(e0, CHUNK)], didx)

        @pl.loop(0, CHUNK, step=16)
        def _(j):
            idx = didx[pl.ds(j, 16)]
            plsc.addupdate_scatter(deg_local, [idx], jnp.ones((16,), jnp.float32))

    pltpu.sync_copy(deg_local, shared.at[s].at[0])
    plsc.subcore_barrier()
    col0 = s * RPT
    pltpu.sync_copy(shared.at[0].at[0].at[pl.ds(col0, RPT)], acc_v)

    @pl.loop(1, NS)
    def _(w):
        pltpu.sync_copy(shared.at[w].at[0].at[pl.ds(col0, RPT)], tmp_v)

        @pl.loop(0, RPT, step=16)
        def _(j):
            acc_v[pl.ds(j, 16)] = acc_v[pl.ds(j, 16)] + tmp_v[pl.ds(j, 16)]

    pltpu.sync_copy(acc_v, out_hbm.at[c].at[0].at[pl.ds(col0, RPT)])


_sc_deg = pl.kernel(
    _sc_deg_body,
    out_type=jax.ShapeDtypeStruct((NC, 1, N_PAD), jnp.float32),
    mesh=plsc.VectorSubcoreMesh(core_axis_name="c", subcore_axis_name="s"),
    scratch_types=[
        pltpu.VMEM((CHUNK,), jnp.int32),
        pltpu.VMEM((N_PAD,), jnp.float32),
        pltpu.VMEM((RPT,), jnp.float32),
        pltpu.VMEM((RPT,), jnp.float32),
        pltpu.VMEM_SHARED((NS, 1, N_PAD), jnp.float32),
    ],
    compiler_params=pltpu.CompilerParams(needs_layout_passes=False),
)


_BLK = 1000
_GRID = N // _BLK


def _leaky(v):
    return jnp.where(v >= 0, v, 0.01 * v)


def _dinv(p0_ref, p1_ref):
    deg = p0_ref[:, 0:1] + p1_ref[:, 0:1] + 1.0
    return lax.rsqrt(deg)


def _tc_pre_body(x_ref, w1_ref, p0_ref, p1_ref, h1p_ref):
    dinv = _dinv(p0_ref, p1_ref)
    h1 = jnp.dot(x_ref[...], w1_ref[...], preferred_element_type=jnp.float32)
    h1p_ref[...] = dinv * h1


def _tc_mid_body(x_ref, q0_ref, q1_ref, h1p_ref, p0_ref, p1_ref, b1_ref,
                 w2a_ref, w2b_ref, h2p_ref):
    dinv = _dinv(p0_ref, p1_ref)
    y = dinv * (q0_ref[...] + q1_ref[...] + h1p_ref[...]) + b1_ref[...]
    h2 = jnp.dot(_leaky(x_ref[...]), w2a_ref[...],
                 preferred_element_type=jnp.float32)
    h2 += jnp.dot(_leaky(y), w2b_ref[...], preferred_element_type=jnp.float32)
    h2p_ref[...] = dinv * h2


def _tc_post_body(r0_ref, r1_ref, h2p_ref, p0_ref, p1_ref, b2_ref, wo_ref,
                  bo_ref, a_ref):
    dinv = _dinv(p0_ref, p1_ref)
    z = _leaky(dinv * (r0_ref[...] + r1_ref[...] + h2p_ref[...]) + b2_ref[...])
    a_ref[...] = jnp.dot(z, wo_ref[...],
                         preferred_element_type=jnp.float32) + bo_ref[...]


def _row_spec(width):
    return pl.BlockSpec((_BLK, width), lambda i: (i, 0))


def _full_spec(shape):
    return pl.BlockSpec(shape, lambda i: (0,) * len(shape))


_tc_pre = pl.pallas_call(
    _tc_pre_body,
    grid=(_GRID,),
    in_specs=[_row_spec(D), _full_spec((D, D)), _row_spec(1), _row_spec(1)],
    out_specs=_row_spec(D),
    out_shape=jax.ShapeDtypeStruct((N, D), jnp.float32),
)

_tc_mid = pl.pallas_call(
    _tc_mid_body,
    grid=(_GRID,),
    in_specs=[_row_spec(D), _row_spec(D), _row_spec(D), _row_spec(D),
              _row_spec(1), _row_spec(1), _full_spec((1, D)),
              _full_spec((D, D)), _full_spec((D, D))],
    out_specs=_row_spec(D),
    out_shape=jax.ShapeDtypeStruct((N, D), jnp.float32),
)

_tc_post = pl.pallas_call(
    _tc_post_body,
    grid=(_GRID,),
    in_specs=[_row_spec(D), _row_spec(D), _row_spec(D),
              _row_spec(1), _row_spec(1), _full_spec((1, D)),
              _full_spec((D, D)), _full_spec((1, D))],
    out_specs=_row_spec(D),
    out_shape=jax.ShapeDtypeStruct((N, D), jnp.float32),
)


def kernel(x, edge_index, W1, b1, W2, b2, Wo, bo):
    src = edge_index[0].astype(jnp.int32)
    dst = edge_index[1].astype(jnp.int32)
    npad = E_PAD - E
    src_p = jnp.concatenate([src, jnp.zeros((npad,), jnp.int32)])
    dst_p = jnp.concatenate([dst, jnp.full((npad,), N, jnp.int32)])
    zeros128 = jnp.zeros((N_PAD, D), jnp.float32)

    degp = _sc_deg(dst_p)
    p0 = degp[0, 0, :N, None]
    p1 = degp[1, 0, :N, None]

    h1p = _tc_pre(x, W1, p0, p1)
    agg1 = _sc_agg128(h1p, src_p, dst_p, zeros128)
    h2p = _tc_mid(x, agg1[0, :N], agg1[1, :N], h1p, p0, p1,
                  b1.reshape(1, D), W2[:D], W2[D:])
    agg2 = _sc_agg128(h2p, src_p, dst_p, zeros128)

    wo_pad = jnp.pad(Wo, ((0, 0), (0, D - 1)))
    bo_pad = jnp.pad(bo, (0, D - 1)).reshape(1, D)
    apad = _tc_post(agg2[0, :N], agg2[1, :N], h2p, p0, p1,
                    b2.reshape(1, D), wo_pad, bo_pad)
    return apad[:, :1]
